# baseline (speedup 1.0000x reference)
import functools

import jax
import jax.numpy as jnp
from jax import lax
from jax.experimental import pallas as pl
from jax.experimental.pallas import tpu as pltpu
from jax.experimental.pallas import tpu_sc as plsc

_NUM_P = 20000
_NUM_G = 500
_GPAD = 512
_LANES = 16

_R_TC = 12000
_R_SC = _NUM_P - _R_TC
_TC_BLK = 2000

_ROWS_BIG = 256
_ROWS_SMALL = 248
_NBIG = 8
_CHUNK = 64
_NFULL = 3
_TAIL_BIG = _ROWS_BIG - _NFULL * _CHUNK
_TAIL_SMALL = _ROWS_SMALL - _NFULL * _CHUNK


def _coef_body(gt_ref, out_ref):
    cx = gt_ref[0:1, :]
    cy = gt_ref[1:2, :]
    w = gt_ref[2:3, :]
    h = gt_ref[3:4, :]
    ang = gt_ref[4:5, :]
    cos = jnp.cos(ang)
    sin = jnp.sin(ang)
    ia = (2.0 / w) ** 2
    ib = (2.0 / h) ** 2
    a_c = cos * cos * ia + sin * sin * ib
    c_c = sin * sin * ia + cos * cos * ib
    b_c = 2.0 * cos * sin * (ia - ib)
    l1 = -(2.0 * a_c * cx + b_c * cy)
    l2 = -(2.0 * c_c * cy + b_c * cx)
    k_c = a_c * cx * cx + c_c * cy * cy + b_c * cx * cy
    out_ref[...] = jnp.zeros((8, _GPAD), jnp.float32)
    out_ref[0:1, 0:_NUM_G] = a_c
    out_ref[1:2, 0:_NUM_G] = c_c
    out_ref[2:3, 0:_NUM_G] = b_c
    out_ref[3:4, 0:_NUM_G] = l1
    out_ref[4:5, 0:_NUM_G] = l2
    out_ref[5:6, 0:_NUM_G] = k_c


def _coef_table(gt_bboxes):
    return pl.pallas_call(
        _coef_body,
        out_shape=jax.ShapeDtypeStruct((8, _GPAD), jnp.float32),
    )(gt_bboxes.T)


def _tc_body(gt_ref, pts_ref, out_ref):
    cx = gt_ref[0:1, :]
    cy = gt_ref[1:2, :]
    w = gt_ref[2:3, :]
    h = gt_ref[3:4, :]
    ang = gt_ref[4:5, :]
    cos = jnp.cos(ang)
    sin = jnp.sin(ang)
    inv_a = 2.0 / w
    inv_b = 2.0 / h
    ca = cos * inv_a
    sa = sin * inv_a
    cb = cos * inv_b
    sb = sin * inv_b
    px = pts_ref[:, 0:1]
    py = pts_ref[:, 1:2]
    dx = px - cx
    dy = py - cy
    ox = ca * dx + sa * dy
    oy = cb * dy - sb * dx
    out_ref[...] = ox * ox + oy * oy


def _tc_map(gt_bboxes, points_top):
    num_gts = gt_bboxes.shape[0]
    grid = (_R_TC // _TC_BLK,)
    return pl.pallas_call(
        _tc_body,
        grid=grid,
        in_specs=[
            pl.BlockSpec((5, num_gts), lambda i: (0, 0)),
            pl.BlockSpec((_TC_BLK, 2), lambda i: (i, 0)),
        ],
        out_specs=pl.BlockSpec((_TC_BLK, num_gts), lambda i: (i, 0)),
        out_shape=jax.ShapeDtypeStruct((_R_TC, num_gts), jnp.float32),
    )(gt_bboxes.T, points_top)


def _sc_body(g_hbm, pts_hbm, out_hbm,
             a_v, c_v, b_v, l1_v, l2_v, k_v, buf_v, pts_s):
    wid = lax.axis_index("s") * 2 + lax.axis_index("c")
    is_big = wid < _NBIG
    row0 = jnp.where(is_big,
                     wid * _ROWS_BIG,
                     _NBIG * _ROWS_BIG + (wid - _NBIG) * _ROWS_SMALL)
    row0 = pl.multiple_of(row0, 8)

    coef_refs = (a_v, c_v, b_v, l1_v, l2_v, k_v)
    for i, ref in enumerate(coef_refs):
        pltpu.sync_copy(g_hbm.at[i], ref)

    @pl.when(is_big)
    def _():
        src = pl.multiple_of(row0 * 2, 8)
        pltpu.sync_copy(pts_hbm.at[pl.ds(src, _ROWS_BIG * 2)],
                        pts_s.at[pl.ds(0, _ROWS_BIG * 2)])

    @pl.when(jnp.logical_not(is_big))
    def _():
        src = pl.multiple_of(row0 * 2, 8)
        pltpu.sync_copy(pts_hbm.at[pl.ds(src, _ROWS_SMALL * 2)],
                        pts_s.at[pl.ds(0, _ROWS_SMALL * 2)])

    chunk_offs = [16 * c for c in range(30)] + [484, 480]

    def do_chunk(i, carry):
        def point_pass(offs):
            coefs = [[ref[pl.ds(o, _LANES)] for ref in coef_refs]
                     for o in offs]

            def body(q, pcarry):
                pv = pts_s[pl.ds((i * _CHUNK + q * 8) * 2, _LANES)]
                for k in range(8):
                    px = pv[2 * k]
                    py = pv[2 * k + 1]
                    px2 = px * px
                    py2 = py * py
                    pxpy = px * py
                    row = q * 8 + k
                    for (av, cv, bv, l1v, l2v, kv), o in zip(coefs, offs):
                        acc = (av * px2 + cv * py2 + bv * pxpy
                               + l1v * px + l2v * py + kv)
                        buf_v[row, pl.ds(o, _LANES)] = acc
                return pcarry

            lax.fori_loop(0, _CHUNK // 8, body, 0)

        for g in range(8):
            point_pass(chunk_offs[4 * g:4 * g + 4])

        @pl.when(i < _NFULL)
        def _():
            start = pl.multiple_of(row0 + i * _CHUNK, 8)
            pltpu.sync_copy(buf_v.at[pl.ds(0, _CHUNK)],
                            out_hbm.at[pl.ds(start, _CHUNK)])

        @pl.when(jnp.logical_and(i == _NFULL, is_big))
        def _():
            start = pl.multiple_of(row0 + _NFULL * _CHUNK, 8)
            pltpu.sync_copy(buf_v.at[pl.ds(0, _TAIL_BIG)],
                            out_hbm.at[pl.ds(start, _TAIL_BIG)])

        @pl.when(jnp.logical_and(i == _NFULL, jnp.logical_not(is_big)))
        def _():
            start = pl.multiple_of(row0 + _NFULL * _CHUNK, 8)
            pltpu.sync_copy(buf_v.at[pl.ds(0, _TAIL_SMALL)],
                            out_hbm.at[pl.ds(start, _TAIL_SMALL)])

        return carry

    lax.fori_loop(0, _NFULL + 1, do_chunk, 0)


def _sc_map(g_tab, points_bottom):
    mesh = plsc.VectorSubcoreMesh(core_axis_name="c", subcore_axis_name="s")
    f = functools.partial(
        pl.kernel,
        mesh=mesh,
        out_type=jax.ShapeDtypeStruct((_R_SC, _NUM_G), jnp.float32),
        scratch_types=[
            pltpu.VMEM((_GPAD,), jnp.float32),
            pltpu.VMEM((_GPAD,), jnp.float32),
            pltpu.VMEM((_GPAD,), jnp.float32),
            pltpu.VMEM((_GPAD,), jnp.float32),
            pltpu.VMEM((_GPAD,), jnp.float32),
            pltpu.VMEM((_GPAD,), jnp.float32),
            pltpu.VMEM((_CHUNK, _NUM_G), jnp.float32),
            pltpu.VMEM((_ROWS_BIG * 2,), jnp.float32),
        ],
    )(_sc_body)
    return f(g_tab, points_bottom.reshape(-1))


def kernel(gt_bboxes, points):
    g_tab = _coef_table(gt_bboxes)
    sc_part = _sc_map(g_tab, points[_R_TC:])
    tc_part = _tc_map(gt_bboxes, points[:_R_TC])
    return jnp.concatenate([tc_part, sc_part], axis=0)


# speedup vs baseline: 2.4537x; 2.4537x over previous
import jax
import jax.numpy as jnp
from jax.experimental import pallas as pl


_BLK = 4000
_ROWS = 8


def _body(gt_ref, pts_ref, out_ref):
    cx = gt_ref[0:1, :]
    cy = gt_ref[1:2, :]
    w = gt_ref[2:3, :]
    h = gt_ref[3:4, :]
    ang = gt_ref[4:5, :]
    cos = jnp.cos(ang)
    sin = jnp.sin(ang)
    inv_a = 2.0 / w
    inv_b = 2.0 / h
    ca = cos * inv_a
    sa = sin * inv_a
    cb = cos * inv_b
    sb = sin * inv_b

    def tile(i, carry):
        r = i * _ROWS
        px = pts_ref[pl.ds(r, _ROWS), 0:1]
        py = pts_ref[pl.ds(r, _ROWS), 1:2]
        dx = px - cx
        dy = py - cy
        ox = ca * dx + sa * dy
        oy = cb * dy - sb * dx
        out_ref[pl.ds(r, _ROWS), :] = ox * ox + oy * oy
        return carry

    jax.lax.fori_loop(0, _BLK // _ROWS, tile, 0, unroll=4)


def kernel(gt_bboxes, points):
    num_gts = gt_bboxes.shape[0]
    num_points = points.shape[0]
    gt_t = gt_bboxes.T
    grid = (num_points // _BLK,)
    return pl.pallas_call(
        _body,
        grid=grid,
        in_specs=[
            pl.BlockSpec((5, num_gts), lambda i: (0, 0)),
            pl.BlockSpec((_BLK, 2), lambda i: (i, 0)),
        ],
        out_specs=pl.BlockSpec((_BLK, num_gts), lambda i: (i, 0)),
        out_shape=jax.ShapeDtypeStruct((num_points, num_gts), jnp.float32),
    )(gt_t, points)


# speedup vs baseline: 4.0404x; 1.6467x over previous
import jax
import jax.numpy as jnp
from jax.experimental import pallas as pl


_BLK = 4000


def _body(gt_ref, pts_ref, out_ref):
    cx = gt_ref[0:1, :]
    cy = gt_ref[1:2, :]
    w = gt_ref[2:3, :]
    h = gt_ref[3:4, :]
    ang = gt_ref[4:5, :]
    cos = jnp.cos(ang)
    sin = jnp.sin(ang)
    inv_a = 2.0 / w
    inv_b = 2.0 / h
    ca = cos * inv_a
    sa = sin * inv_a
    cb = cos * inv_b
    sb = sin * inv_b
    px = pts_ref[:, 0:1]
    py = pts_ref[:, 1:2]
    dx = px - cx
    dy = py - cy
    ox = ca * dx + sa * dy
    oy = cb * dy - sb * dx
    out_ref[...] = ox * ox + oy * oy


def kernel(gt_bboxes, points):
    num_gts = gt_bboxes.shape[0]
    num_points = points.shape[0]
    gt_t = gt_bboxes.T
    grid = (num_points // _BLK,)
    return pl.pallas_call(
        _body,
        grid=grid,
        in_specs=[
            pl.BlockSpec((5, num_gts), lambda i: (0, 0)),
            pl.BlockSpec((_BLK, 2), lambda i: (i, 0)),
        ],
        out_specs=pl.BlockSpec((_BLK, num_gts), lambda i: (i, 0)),
        out_shape=jax.ShapeDtypeStruct((num_points, num_gts), jnp.float32),
    )(gt_t, points)
